# baseline (speedup 1.0000x reference)
import functools

import jax
import jax.numpy as jnp
from jax.experimental import pallas as pl
from jax.experimental.pallas import tpu as pltpu
from jax.experimental.pallas import tpu_sc as plsc

VOCAB = 100000
EMB = 128
HID = 128
B = 1024
T = 200
G = 3 * HID

_GW = 128
_KBUF = 5
_NWORK = 32


def _sc_gather(table, idx):
    n = idx.shape[1]
    n_win = n // _GW
    nw = n_win // _NWORK
    idx4d = idx.reshape(_NWORK, nw, 1, _GW)
    mesh = plsc.VectorSubcoreMesh(core_axis_name="c", subcore_axis_name="s")

    @pl.kernel(
        out_type=jax.ShapeDtypeStruct((n, EMB), table.dtype),
        mesh=mesh,
        scratch_types=[
            pltpu.VMEM((nw, 1, _GW), jnp.int32),
            pltpu.VMEM((_KBUF, _GW, EMB), jnp.float32),
            pltpu.SemaphoreType.DMA,
            pltpu.SemaphoreType.DMA,
        ],
    )
    def k(tbl_hbm, idx_hbm, out_hbm, idx_v, rows_v, gsem, ssem):
        wid = jax.lax.axis_index("s") * 2 + jax.lax.axis_index("c")
        wbase = wid * nw
        pltpu.sync_copy(idx_hbm.at[wid], idx_v)

        @pl.loop(0, nw, step=_KBUF)
        def _(w0):
            gs = [
                pltpu.async_copy(
                    tbl_hbm.at[idx_v.at[w0 + b, 0]], rows_v.at[b], gsem)
                for b in range(_KBUF)
            ]
            ss = []
            for b in range(_KBUF):
                gs[b].wait()
                ss.append(pltpu.async_copy(
                    rows_v.at[b],
                    out_hbm.at[pl.ds((wbase + w0 + b) * _GW, _GW)],
                    ssem))
            for s_ in ss:
                s_.wait()

    return k(table, idx4d)


_C = 5
_TCH = T // _C
_TS = 8
_NSTEP = _TCH // _TS
_NB = 2
_BB = B // _NB


def _gru_chunk_body(first, *refs):
    if first:
        emb_ref, wcat_ref, bias_ref, out_ref, hout_ref, h_ref = refs
        hin_ref = None
    else:
        (emb_ref, wcat_ref, bias_ref, hin_ref, _outprev,
         out_ref, hout_ref, h_ref) = refs
    t = pl.program_id(1)

    @pl.when(t == 0)
    def _init():
        if first:
            h_ref[...] = jnp.zeros_like(h_ref)
        else:
            h_ref[...] = hin_ref[...]

    h = h_ref[...]
    wcat = wcat_ref[...]
    bias = bias_ref[...]
    for s in range(_TS):
        eh = jnp.concatenate([emb_ref[s], h], axis=1)
        g = jnp.dot(eh, wcat, preferred_element_type=jnp.float32) + bias
        ur = jnp.tanh(g[:, :HID])
        uz = jnp.tanh(g[:, HID:2 * HID])
        ch = g[:, 3 * HID:]
        nn = jnp.tanh(g[:, 2 * HID:3 * HID] + ch + ur * ch)
        d = h - nn
        h = nn + 0.5 * (d + uz * d)
        out_ref[:, s, :] = h
    h_ref[...] = h
    hout_ref[...] = h


_OUT_SHAPES = [
    jax.ShapeDtypeStruct((B, T, HID), jnp.float32),
    jax.ShapeDtypeStruct((B, HID), jnp.float32),
]
_CPARAMS = pltpu.CompilerParams(dimension_semantics=("arbitrary", "arbitrary"))


def _scan_chunk(c, emb_c, wcat, bias, h_in, out_sofar):
    first = c == 0
    base_specs = [
        pl.BlockSpec((_TS, _BB, EMB), lambda j, t: (t, j, 0)),
        pl.BlockSpec((EMB + HID, 4 * HID), lambda j, t: (0, 0)),
        pl.BlockSpec((1, 4 * HID), lambda j, t: (0, 0)),
    ]
    out_specs = [
        pl.BlockSpec((_BB, _TS, HID), lambda j, t: (j, t + c * _NSTEP, 0)),
        pl.BlockSpec((_BB, HID), lambda j, t: (j, 0)),
    ]
    if first:
        return pl.pallas_call(
            functools.partial(_gru_chunk_body, True),
            grid=(_NB, _NSTEP),
            in_specs=base_specs,
            out_specs=out_specs,
            out_shape=_OUT_SHAPES,
            scratch_shapes=[pltpu.VMEM((_BB, HID), jnp.float32)],
            compiler_params=_CPARAMS,
        )(emb_c, wcat, bias)
    return pl.pallas_call(
        functools.partial(_gru_chunk_body, False),
        grid=(_NB, _NSTEP),
        in_specs=base_specs + [
            pl.BlockSpec((_BB, HID), lambda j, t: (j, 0)),
            pl.BlockSpec(memory_space=pl.ANY),
        ],
        out_specs=out_specs,
        out_shape=_OUT_SHAPES,
        scratch_shapes=[pltpu.VMEM((_BB, HID), jnp.float32)],
        input_output_aliases={4: 0},
        compiler_params=_CPARAMS,
    )(emb_c, wcat, bias, h_in, out_sofar)


def kernel(x, table, W_ih, W_hh, b_ih, b_hh):
    idx = x.astype(jnp.int32).T.reshape(1, T * B)
    z_eh = jnp.zeros((EMB, HID), jnp.float32)
    z_hh = jnp.zeros((HID, HID), jnp.float32)
    top = jnp.concatenate(
        [0.5 * W_ih[:, :2 * HID], W_ih[:, 2 * HID:], z_eh], axis=1)
    bot = jnp.concatenate(
        [0.5 * W_hh[:, :2 * HID], z_hh, 0.5 * W_hh[:, 2 * HID:]], axis=1)
    wcat = jnp.concatenate([top, bot], axis=0)
    bias = jnp.concatenate(
        [0.5 * (b_ih + b_hh)[:2 * HID], b_ih[2 * HID:],
         0.5 * b_hh[2 * HID:]]).reshape(1, 4 * HID)
    embs = [
        _sc_gather(table, idx[:, c * _TCH * B:(c + 1) * _TCH * B])
        .reshape(_TCH, B, EMB)
        for c in range(_C)
    ]
    out, h = _scan_chunk(0, embs[0], wcat, bias, None, None)
    for c in range(1, _C):
        out, h = _scan_chunk(c, embs[c], wcat, bias, h, out)
    return out


# speedup vs baseline: 1.0280x; 1.0280x over previous
import functools

import jax
import jax.numpy as jnp
from jax.experimental import pallas as pl
from jax.experimental.pallas import tpu as pltpu
from jax.experimental.pallas import tpu_sc as plsc

VOCAB = 100000
EMB = 128
HID = 128
B = 1024
T = 200
G = 3 * HID

_GATHER_WINDOW = 128


def _sc_gather(table, idx):
    n = idx.shape[1]
    mesh = plsc.VectorSubcoreMesh(core_axis_name="c", subcore_axis_name="s")

    @pl.kernel(
        out_type=jax.ShapeDtypeStruct((n, EMB), table.dtype),
        mesh=mesh,
    )
    def k(tbl_hbm, idx_hbm, out_hbm):
        def body(i_vmem, o_vmem):
            pltpu.sync_copy(tbl_hbm.at[i_vmem.at[0]], o_vmem)

        pltpu.emit_pipeline(
            body,
            grid=(n // _GATHER_WINDOW,),
            in_specs=[
                pl.BlockSpec((1, _GATHER_WINDOW), lambda i: (0, i)),
            ],
            out_specs=[
                pl.BlockSpec((_GATHER_WINDOW, EMB), lambda i: (i, 0)),
            ],
            core_axis_name=("c", "s"),
            dimension_semantics=(pltpu.PARALLEL,),
        )(idx_hbm, out_hbm)

    return k(table, idx)


_C = 5
_TCH = T // _C
_TS = 8
_NSTEP = _TCH // _TS
_NB = 2
_BB = B // _NB


def _gru_chunk_body(first, *refs):
    if first:
        emb_ref, wcat_ref, bias_ref, out_ref, hout_ref, h_ref = refs
        hin_ref = None
    else:
        (emb_ref, wcat_ref, bias_ref, hin_ref, _outprev,
         out_ref, hout_ref, h_ref) = refs
    t = pl.program_id(1)

    @pl.when(t == 0)
    def _init():
        if first:
            h_ref[...] = jnp.zeros_like(h_ref)
        else:
            h_ref[...] = hin_ref[...]

    h = h_ref[...]
    wcat = wcat_ref[...]
    bias = bias_ref[...]
    for s in range(_TS):
        eh = jnp.concatenate([emb_ref[s], h], axis=1)
        g = jnp.dot(eh, wcat, preferred_element_type=jnp.float32) + bias
        ur = jnp.tanh(g[:, :HID])
        uz = jnp.tanh(g[:, HID:2 * HID])
        ch = g[:, 3 * HID:]
        nn = jnp.tanh(g[:, 2 * HID:3 * HID] + ch + ur * ch)
        d = h - nn
        h = nn + 0.5 * (d + uz * d)
        out_ref[:, s, :] = h
    h_ref[...] = h
    hout_ref[...] = h


_OUT_SHAPES = [
    jax.ShapeDtypeStruct((B, T, HID), jnp.float32),
    jax.ShapeDtypeStruct((B, HID), jnp.float32),
]
_CPARAMS = pltpu.CompilerParams(dimension_semantics=("arbitrary", "arbitrary"))


def _scan_chunk(c, emb_c, wcat, bias, h_in, out_sofar):
    first = c == 0
    base_specs = [
        pl.BlockSpec((_TS, _BB, EMB), lambda j, t: (t, j, 0)),
        pl.BlockSpec((EMB + HID, 4 * HID), lambda j, t: (0, 0)),
        pl.BlockSpec((1, 4 * HID), lambda j, t: (0, 0)),
    ]
    out_specs = [
        pl.BlockSpec((_BB, _TS, HID), lambda j, t: (j, t + c * _NSTEP, 0)),
        pl.BlockSpec((_BB, HID), lambda j, t: (j, 0)),
    ]
    if first:
        return pl.pallas_call(
            functools.partial(_gru_chunk_body, True),
            grid=(_NB, _NSTEP),
            in_specs=base_specs,
            out_specs=out_specs,
            out_shape=_OUT_SHAPES,
            scratch_shapes=[pltpu.VMEM((_BB, HID), jnp.float32)],
            compiler_params=_CPARAMS,
        )(emb_c, wcat, bias)
    return pl.pallas_call(
        functools.partial(_gru_chunk_body, False),
        grid=(_NB, _NSTEP),
        in_specs=base_specs + [
            pl.BlockSpec((_BB, HID), lambda j, t: (j, 0)),
            pl.BlockSpec(memory_space=pl.ANY),
        ],
        out_specs=out_specs,
        out_shape=_OUT_SHAPES,
        scratch_shapes=[pltpu.VMEM((_BB, HID), jnp.float32)],
        input_output_aliases={4: 0},
        compiler_params=_CPARAMS,
    )(emb_c, wcat, bias, h_in, out_sofar)


def kernel(x, table, W_ih, W_hh, b_ih, b_hh):
    idx = x.astype(jnp.int32).T.reshape(1, T * B)
    z_eh = jnp.zeros((EMB, HID), jnp.float32)
    z_hh = jnp.zeros((HID, HID), jnp.float32)
    top = jnp.concatenate(
        [0.5 * W_ih[:, :2 * HID], W_ih[:, 2 * HID:], z_eh], axis=1)
    bot = jnp.concatenate(
        [0.5 * W_hh[:, :2 * HID], z_hh, 0.5 * W_hh[:, 2 * HID:]], axis=1)
    wcat = jnp.concatenate([top, bot], axis=0)
    bias = jnp.concatenate(
        [0.5 * (b_ih + b_hh)[:2 * HID], b_ih[2 * HID:],
         0.5 * b_hh[2 * HID:]]).reshape(1, 4 * HID)
    embs = [
        _sc_gather(table, idx[:, c * _TCH * B:(c + 1) * _TCH * B])
        .reshape(_TCH, B, EMB)
        for c in range(_C)
    ]
    out, h = _scan_chunk(0, embs[0], wcat, bias, None, None)
    for c in range(1, _C):
        out, h = _scan_chunk(c, embs[c], wcat, bias, h, out)
    return out
